# baseline (speedup 1.0000x reference)
import functools

import jax
import jax.numpy as jnp
from jax import lax
from jax.experimental import pallas as pl
from jax.experimental.pallas import tpu as pltpu
from jax.experimental.pallas import tpu_sc as plsc

N = 10000
E = 320000
D = 128
H = 32
G = 64

NC = 2
NS = 16
NW = NC * NS
LANE = 128
NPAD = 10240
EP = 327680
EPB = EP // LANE
RPW = EPB // NW
KG = 8
RPT = NPAD // NS

_SELU_SCALE = 1.0507009873554805
_SELU_ALPHA = 1.6732632423543772

_mesh = plsc.VectorSubcoreMesh(
    core_axis_name="c", subcore_axis_name="s", num_cores=NC, num_subcores=NS)
_sc_params = pltpu.CompilerParams(use_tc_tiling_on_sc=False,
                                  skip_device_barrier=True)


@functools.partial(
    pl.kernel,
    out_type=jax.ShapeDtypeStruct((NC * NPAD,), jnp.float32),
    mesh=_mesh,
    scratch_types=[
        pltpu.VMEM((2, KG, LANE), jnp.int32),
        pltpu.VMEM((LANE,), jnp.float32),
        pltpu.VMEM((RPT,), jnp.float32),
        pltpu.VMEM_SHARED((NPAD,), jnp.float32),
        pltpu.SemaphoreType.DMA,
        pltpu.SemaphoreType.DMA,
    ],
    compiler_params=_sc_params,
)
def _sc_deg(colp_hbm, deg_hbm, idx_v, ones_v, stage_v, deg_sh, ssem0, ssem1):
    c = lax.axis_index("c")
    s = lax.axis_index("s")
    wid = c * NS + s
    ssems = (ssem0, ssem1)
    zero16 = jnp.zeros((16,), jnp.float32)
    one16 = jnp.ones((16,), jnp.float32)

    def _fill_stage(i, carry):
        stage_v[pl.ds(i * 16, 16)] = zero16
        return carry

    lax.fori_loop(0, RPT // 16, _fill_stage, 0)

    def _fill_ones(i, carry):
        ones_v[pl.ds(i * 16, 16)] = one16
        return carry

    lax.fori_loop(0, LANE // 16, _fill_ones, 0)

    pltpu.sync_copy(stage_v, deg_sh.at[pl.ds(s * RPT, RPT)])
    plsc.subcore_barrier()

    def _loadc(p, g):
        gbase = wid * RPW + g * KG
        pltpu.sync_copy(colp_hbm.at[pl.ds(gbase, KG)], idx_v.at[p])

    def _scatter(p):
        return [
            pltpu.async_copy(ones_v, deg_sh.at[idx_v.at[p, j]], ssems[p],
                             add=True)
            for j in range(KG)
        ]

    _loadc(0, 0)
    pend = {0: [], 1: []}
    for g in range(RPW // KG):
        p = g & 1
        q = 1 - p
        if g + 1 < RPW // KG:
            for dsc in pend[q]:
                dsc.wait()
            pend[q] = []
            _loadc(q, g + 1)
        pend[p] = _scatter(p)
    for p in (0, 1):
        for dsc in pend[p]:
            dsc.wait()
    plsc.subcore_barrier()

    pltpu.sync_copy(deg_sh.at[pl.ds(s * RPT, RPT)], stage_v)
    pltpu.sync_copy(stage_v, deg_hbm.at[pl.ds(c * NPAD + s * RPT, RPT)])


_NGC = RPW // KG
_CE = KG * LANE


@functools.partial(
    pl.kernel,
    out_type=jax.ShapeDtypeStruct((NC * NPAD, H), jnp.float32),
    mesh=_mesh,
    scratch_types=[
        pltpu.VMEM((2, _CE), jnp.int32),
        pltpu.VMEM((2, KG, LANE), jnp.int32),
        pltpu.VMEM((2, _CE, H), jnp.float32),
        pltpu.VMEM((RPT, H), jnp.float32),
        pltpu.VMEM_SHARED((NPAD, H), jnp.float32),
        pltpu.VMEM_SHARED((NPAD, H), jnp.float32),
        pltpu.SemaphoreType.DMA,
        pltpu.SemaphoreType.DMA,
        pltpu.SemaphoreType.DMA,
        pltpu.SemaphoreType.DMA,
    ],
    compiler_params=_sc_params,
)
def _sc_agg(rowf_hbm, colp_hbm, hp_hbm, agg_hbm,
            idxg, idxs, rows, stage, agg_sh, hp_sh,
            gsem0, gsem1, ssem0, ssem1):
    c = lax.axis_index("c")
    s = lax.axis_index("s")
    wid = c * NS + s
    gsems = (gsem0, gsem1)
    ssems = (ssem0, ssem1)
    zero16 = jnp.zeros((16,), jnp.float32)

    def _zrow(r, carry):
        stage[r, pl.ds(0, 16)] = zero16
        stage[r, pl.ds(16, 16)] = zero16
        return carry

    lax.fori_loop(0, RPT, _zrow, 0)
    pltpu.sync_copy(stage, agg_sh.at[pl.ds(s * RPT, RPT)])
    pltpu.sync_copy(hp_hbm.at[pl.ds(s * RPT, RPT)],
                    hp_sh.at[pl.ds(s * RPT, RPT)])
    plsc.subcore_barrier()

    def _load(p, g):
        gbase = wid * RPW + g * KG
        pltpu.sync_copy(rowf_hbm.at[pl.ds(gbase * LANE, _CE)], idxg.at[p])
        pltpu.sync_copy(colp_hbm.at[pl.ds(gbase, KG)], idxs.at[p])

    def _gather(p, g):
        src = hp_sh if (g & 1) == 0 else hp_hbm
        return pltpu.async_copy(src.at[idxg.at[p]], rows.at[p], gsems[p])

    def _scatter(p):
        return [
            pltpu.async_copy(rows.at[p, pl.ds(j * LANE, LANE)],
                             agg_sh.at[idxs.at[p, j]], ssems[p], add=True)
            for j in range(KG)
        ]

    _load(0, 0)
    gd = {0: _gather(0, 0)}
    sc_pending = {0: [], 1: []}
    for g in range(_NGC):
        p = g & 1
        q = 1 - p
        if g + 1 < _NGC:
            for dsc in sc_pending[q]:
                dsc.wait()
            sc_pending[q] = []
            _load(q, g + 1)
            gd[q] = _gather(q, g + 1)
        gd[p].wait()
        sc_pending[p] = _scatter(p)
    for p in (0, 1):
        for dsc in sc_pending[p]:
            dsc.wait()
    plsc.subcore_barrier()

    pltpu.sync_copy(agg_sh.at[pl.ds(s * RPT, RPT)], stage)
    pltpu.sync_copy(stage, agg_hbm.at[pl.ds(c * NPAD + s * RPT, RPT)])


_BLK = 1024


def _tc_hp_body(x_ref, w_ref, deg_ref, hp_ref):
    dp = deg_ref[...]
    dinv = lax.rsqrt(dp[:, 0:1] + dp[:, 1:2] + 1.0)
    h = jnp.dot(x_ref[...], w_ref[...], preferred_element_type=jnp.float32)
    hp_ref[...] = h * dinv


def _tc_hp(xpad, w, deg_t):
    return pl.pallas_call(
        _tc_hp_body,
        grid=(NPAD // _BLK,),
        in_specs=[
            pl.BlockSpec((_BLK, D), lambda i: (i, 0)),
            pl.BlockSpec((D, H), lambda i: (0, 0)),
            pl.BlockSpec((_BLK, NC), lambda i: (i, 0)),
        ],
        out_specs=pl.BlockSpec((_BLK, H), lambda i: (i, 0)),
        out_shape=jax.ShapeDtypeStruct((NPAD, H), jnp.float32),
    )(xpad, w, deg_t)


def _selu(v):
    return _SELU_SCALE * jnp.where(
        v > 0, v, _SELU_ALPHA * (jnp.exp(jnp.minimum(v, 0.0)) - 1.0))


def _tc_final_body(agg0, agg1, hp, deg_ref, bgcn, batch2, w1, b1, w2, b2,
                   out_ref, acc):
    i = pl.program_id(0)

    @pl.when(i == 0)
    def _():
        acc[...] = jnp.zeros_like(acc)

    dp = deg_ref[...]
    dinv = lax.rsqrt(dp[:, 0:1] + dp[:, 1:2] + 1.0)
    o = (agg0[...] + agg1[...] + hp[...]) * dinv + bgcn[...]
    o = _selu(o)
    bb = batch2[...]
    mt = (lax.broadcasted_iota(jnp.int32, (G, _BLK), 0) == bb).astype(
        jnp.float32)
    acc[...] += jnp.dot(mt, o, preferred_element_type=jnp.float32, precision=lax.Precision.HIGHEST)

    @pl.when(i == NPAD // _BLK - 1)
    def _():
        z = _selu(jnp.dot(acc[...], w1[...],
                          preferred_element_type=jnp.float32) + b1[...])
        out_ref[...] = jnp.dot(z, w2[...],
                               preferred_element_type=jnp.float32) + b2[...]


def _tc_final(agg0, agg1, hp, deg_t, bgcn, batch2, w1, b1, w2, b2):
    return pl.pallas_call(
        _tc_final_body,
        grid=(NPAD // _BLK,),
        in_specs=[
            pl.BlockSpec((_BLK, H), lambda i: (i, 0)),
            pl.BlockSpec((_BLK, H), lambda i: (i, 0)),
            pl.BlockSpec((_BLK, H), lambda i: (i, 0)),
            pl.BlockSpec((_BLK, NC), lambda i: (i, 0)),
            pl.BlockSpec((1, H), lambda i: (0, 0)),
            pl.BlockSpec((1, _BLK), lambda i: (0, i)),
            pl.BlockSpec((H, H), lambda i: (0, 0)),
            pl.BlockSpec((1, H), lambda i: (0, 0)),
            pl.BlockSpec((H, 1), lambda i: (0, 0)),
            pl.BlockSpec((1, 1), lambda i: (0, 0)),
        ],
        out_specs=pl.BlockSpec((G, 1), lambda i: (0, 0)),
        out_shape=jax.ShapeDtypeStruct((G, 1), jnp.float32),
        scratch_shapes=[pltpu.VMEM((G, H), jnp.float32)],
    )(agg0, agg1, hp, deg_t, bgcn, batch2, w1, b1, w2, b2)


def kernel(x, edge_index, batch, W_gcn, b_gcn, W1, b1, W2, b2):
    pad = jnp.full((EP - E,), N, jnp.int32)
    rowf = jnp.concatenate([edge_index[0], pad])
    colp = jnp.concatenate([edge_index[1], pad]).reshape(EPB, LANE)

    deg_flat = _sc_deg(colp)
    deg_t = deg_flat.reshape(NC, NPAD).T

    xpad = jnp.concatenate([x, jnp.zeros((NPAD - N, D), jnp.float32)])
    hp = _tc_hp(xpad, W_gcn, deg_t)

    agg = _sc_agg(rowf, colp, hp)

    batch2 = jnp.concatenate(
        [batch, jnp.full((NPAD - N,), G, jnp.int32)]).reshape(1, NPAD)
    return _tc_final(agg[:NPAD], agg[NPAD:], hp, deg_t,
                     b_gcn.reshape(1, H), batch2,
                     W1, b1.reshape(1, H), W2, b2.reshape(1, 1))


# speedup vs baseline: 1.1971x; 1.1971x over previous
import functools

import jax
import jax.numpy as jnp
from jax import lax
from jax.experimental import pallas as pl
from jax.experimental.pallas import tpu as pltpu
from jax.experimental.pallas import tpu_sc as plsc

N = 10000
E = 320000
D = 128
H = 32
G = 64

NC = 2
NS = 16
NW = NC * NS
LANE = 128
NPAD = 10240
EP = 327680
EPB = EP // LANE
RPW = EPB // NW
KG = 8
RPT = NPAD // NS

_SELU_SCALE = 1.0507009873554805
_SELU_ALPHA = 1.6732632423543772

_mesh = plsc.VectorSubcoreMesh(
    core_axis_name="c", subcore_axis_name="s", num_cores=NC, num_subcores=NS)
_sc_params = pltpu.CompilerParams(use_tc_tiling_on_sc=False,
                                  skip_device_barrier=True)


@functools.partial(
    pl.kernel,
    out_type=jax.ShapeDtypeStruct((NC * NPAD,), jnp.float32),
    mesh=_mesh,
    scratch_types=[
        pltpu.VMEM((2, KG, LANE), jnp.int32),
        pltpu.VMEM((LANE,), jnp.float32),
        pltpu.VMEM((RPT,), jnp.float32),
        pltpu.VMEM_SHARED((NPAD,), jnp.float32),
        pltpu.SemaphoreType.DMA,
        pltpu.SemaphoreType.DMA,
    ],
    compiler_params=_sc_params,
)
def _sc_deg(colp_hbm, deg_hbm, idx_v, ones_v, stage_v, deg_sh, ssem0, ssem1):
    c = lax.axis_index("c")
    s = lax.axis_index("s")
    wid = c * NS + s
    ssems = (ssem0, ssem1)
    zero16 = jnp.zeros((16,), jnp.float32)
    one16 = jnp.ones((16,), jnp.float32)

    def _fill_stage(i, carry):
        stage_v[pl.ds(i * 16, 16)] = zero16
        return carry

    lax.fori_loop(0, RPT // 16, _fill_stage, 0)

    def _fill_ones(i, carry):
        ones_v[pl.ds(i * 16, 16)] = one16
        return carry

    lax.fori_loop(0, LANE // 16, _fill_ones, 0)

    pltpu.sync_copy(stage_v, deg_sh.at[pl.ds(s * RPT, RPT)])
    plsc.subcore_barrier()

    def _loadc(p, g):
        gbase = wid * RPW + g * KG
        pltpu.sync_copy(colp_hbm.at[pl.ds(gbase, KG)], idx_v.at[p])

    def _scatter(p):
        return [
            pltpu.async_copy(ones_v, deg_sh.at[idx_v.at[p, j]], ssems[p],
                             add=True)
            for j in range(KG)
        ]

    _loadc(0, 0)
    pend = {0: [], 1: []}
    for g in range(RPW // KG):
        p = g & 1
        q = 1 - p
        if g + 1 < RPW // KG:
            for dsc in pend[q]:
                dsc.wait()
            pend[q] = []
            _loadc(q, g + 1)
        pend[p] = _scatter(p)
    for p in (0, 1):
        for dsc in pend[p]:
            dsc.wait()
    plsc.subcore_barrier()

    pltpu.sync_copy(deg_sh.at[pl.ds(s * RPT, RPT)], stage_v)
    pltpu.sync_copy(stage_v, deg_hbm.at[pl.ds(c * NPAD + s * RPT, RPT)])


_NGC = RPW // KG
_CE = KG * LANE


@functools.partial(
    pl.kernel,
    out_type=jax.ShapeDtypeStruct((NC * NPAD, H), jnp.float32),
    mesh=_mesh,
    scratch_types=[
        pltpu.VMEM((2, _CE), jnp.int32),
        pltpu.VMEM((2, KG, LANE), jnp.int32),
        pltpu.VMEM((2, _CE, H), jnp.float32),
        pltpu.VMEM((RPT, H), jnp.float32),
        pltpu.VMEM_SHARED((NPAD, H), jnp.float32),
        pltpu.VMEM_SHARED((NPAD, H), jnp.float32),
        pltpu.SemaphoreType.DMA,
        pltpu.SemaphoreType.DMA,
        pltpu.SemaphoreType.DMA,
        pltpu.SemaphoreType.DMA,
    ],
    compiler_params=_sc_params,
)
def _sc_agg(rowf_hbm, colp_hbm, hp_hbm, agg_hbm,
            idxg, idxs, rows, stage, agg_sh, hp_sh,
            gsem0, gsem1, ssem0, ssem1):
    c = lax.axis_index("c")
    s = lax.axis_index("s")
    wid = c * NS + s
    gsems = (gsem0, gsem1)
    ssems = (ssem0, ssem1)
    zero16 = jnp.zeros((16,), jnp.float32)

    def _zrow(r, carry):
        stage[r, pl.ds(0, 16)] = zero16
        stage[r, pl.ds(16, 16)] = zero16
        return carry

    lax.fori_loop(0, RPT, _zrow, 0)
    pltpu.sync_copy(stage, agg_sh.at[pl.ds(s * RPT, RPT)])
    pltpu.sync_copy(hp_hbm.at[pl.ds(s * RPT, RPT)],
                    hp_sh.at[pl.ds(s * RPT, RPT)])
    plsc.subcore_barrier()

    def _load(p, g):
        gbase = wid * RPW + g * KG
        pltpu.sync_copy(rowf_hbm.at[pl.ds(gbase * LANE, _CE)], idxg.at[p])
        pltpu.sync_copy(colp_hbm.at[pl.ds(gbase, KG)], idxs.at[p])

    def _gather(p, g):
        del g
        return pltpu.async_copy(hp_sh.at[idxg.at[p]], rows.at[p], gsems[p])

    def _scatter(p):
        return [
            pltpu.async_copy(rows.at[p, pl.ds(j * LANE, LANE)],
                             agg_sh.at[idxs.at[p, j]], ssems[p], add=True)
            for j in range(KG)
        ]

    _load(0, 0)
    gd = {0: _gather(0, 0)}
    sc_pending = {0: [], 1: []}
    for g in range(_NGC):
        p = g & 1
        q = 1 - p
        if g + 1 < _NGC:
            for dsc in sc_pending[q]:
                dsc.wait()
            sc_pending[q] = []
            _load(q, g + 1)
            gd[q] = _gather(q, g + 1)
        gd[p].wait()
        sc_pending[p] = _scatter(p)
    for p in (0, 1):
        for dsc in sc_pending[p]:
            dsc.wait()
    plsc.subcore_barrier()

    pltpu.sync_copy(agg_sh.at[pl.ds(s * RPT, RPT)], stage)
    pltpu.sync_copy(stage, agg_hbm.at[pl.ds(c * NPAD + s * RPT, RPT)])


_BLK = 2048


def _tc_hp_body(x_ref, w_ref, deg_ref, hp_ref):
    dp = deg_ref[...]
    dinv = lax.rsqrt(dp[:, 0:1] + dp[:, 1:2] + 1.0)
    h = jnp.dot(x_ref[...], w_ref[...], preferred_element_type=jnp.float32)
    hp_ref[...] = h * dinv


def _tc_hp(xpad, w, deg_t):
    return pl.pallas_call(
        _tc_hp_body,
        grid=(NPAD // _BLK,),
        in_specs=[
            pl.BlockSpec((_BLK, D), lambda i: (i, 0)),
            pl.BlockSpec((D, H), lambda i: (0, 0)),
            pl.BlockSpec((_BLK, NC), lambda i: (i, 0)),
        ],
        out_specs=pl.BlockSpec((_BLK, H), lambda i: (i, 0)),
        out_shape=jax.ShapeDtypeStruct((NPAD, H), jnp.float32),
    )(xpad, w, deg_t)


def _selu(v):
    return _SELU_SCALE * jnp.where(
        v > 0, v, _SELU_ALPHA * (jnp.exp(jnp.minimum(v, 0.0)) - 1.0))


def _tc_final_body(agg0, agg1, hp, deg_ref, bgcn, batch2, w1, b1, w2, b2,
                   out_ref, acc):
    i = pl.program_id(0)

    @pl.when(i == 0)
    def _():
        acc[...] = jnp.zeros_like(acc)

    dp = deg_ref[...]
    dinv = lax.rsqrt(dp[:, 0:1] + dp[:, 1:2] + 1.0)
    o = (agg0[...] + agg1[...] + hp[...]) * dinv + bgcn[...]
    o = _selu(o)
    bb = batch2[...]
    mt = (lax.broadcasted_iota(jnp.int32, (G, _BLK), 0) == bb).astype(
        jnp.float32)
    acc[...] += jnp.dot(mt, o, preferred_element_type=jnp.float32, precision=lax.Precision.HIGHEST)

    @pl.when(i == NPAD // _BLK - 1)
    def _():
        z = _selu(jnp.dot(acc[...], w1[...],
                          preferred_element_type=jnp.float32) + b1[...])
        out_ref[...] = jnp.dot(z, w2[...],
                               preferred_element_type=jnp.float32) + b2[...]


def _tc_final(agg0, agg1, hp, deg_t, bgcn, batch2, w1, b1, w2, b2):
    return pl.pallas_call(
        _tc_final_body,
        grid=(NPAD // _BLK,),
        in_specs=[
            pl.BlockSpec((_BLK, H), lambda i: (i, 0)),
            pl.BlockSpec((_BLK, H), lambda i: (i, 0)),
            pl.BlockSpec((_BLK, H), lambda i: (i, 0)),
            pl.BlockSpec((_BLK, NC), lambda i: (i, 0)),
            pl.BlockSpec((1, H), lambda i: (0, 0)),
            pl.BlockSpec((1, _BLK), lambda i: (0, i)),
            pl.BlockSpec((H, H), lambda i: (0, 0)),
            pl.BlockSpec((1, H), lambda i: (0, 0)),
            pl.BlockSpec((H, 1), lambda i: (0, 0)),
            pl.BlockSpec((1, 1), lambda i: (0, 0)),
        ],
        out_specs=pl.BlockSpec((G, 1), lambda i: (0, 0)),
        out_shape=jax.ShapeDtypeStruct((G, 1), jnp.float32),
        scratch_shapes=[pltpu.VMEM((G, H), jnp.float32)],
    )(agg0, agg1, hp, deg_t, bgcn, batch2, w1, b1, w2, b2)


def kernel(x, edge_index, batch, W_gcn, b_gcn, W1, b1, W2, b2):
    pad = jnp.full((EP - E,), N, jnp.int32)
    rowf = jnp.concatenate([edge_index[0], pad])
    colp = jnp.concatenate([edge_index[1], pad]).reshape(EPB, LANE)

    deg_flat = _sc_deg(colp)
    deg_t = deg_flat.reshape(NC, NPAD).T

    xpad = jnp.concatenate([x, jnp.zeros((NPAD - N, D), jnp.float32)])
    hp = _tc_hp(xpad, W_gcn, deg_t)

    agg = _sc_agg(rowf, colp, hp)

    batch2 = jnp.concatenate(
        [batch, jnp.full((NPAD - N,), G, jnp.int32)]).reshape(1, NPAD)
    return _tc_final(agg[:NPAD], agg[NPAD:], hp, deg_t,
                     b_gcn.reshape(1, H), batch2,
                     W1, b1.reshape(1, H), W2, b2.reshape(1, 1))
